# Initial kernel scaffold; baseline (speedup 1.0000x reference)
import jax
import jax.numpy as jnp
from jax.experimental import pallas as pl


def kernel(field_0, field_1, field_2, field_3, field_4, field_5, field_6, field_7, table_0, table_1, table_2, table_3, table_4, table_5, table_6, table_7, W, b):
    raise NotImplementedError("write your pallas kernel here")


# speedup vs baseline: 1.8224x; 1.8224x over previous
import functools

import jax
import jax.numpy as jnp
from jax import lax
from jax.experimental import pallas as pl
from jax.experimental.pallas import tpu as pltpu
from jax.experimental.pallas import tpu_sc as plsc

NUM_FIELDS = 8
VOCAB = 100000
EMB = 32
BATCH = 16384
ROUTING_DIM = 128

NC, NS = 2, 16
NW = NC * NS
CHUNK = 128
B_PER_W = BATCH // NW
N_CHUNKS = B_PER_W // CHUNK

@functools.lru_cache(maxsize=1)
def _make_sc_gather():
    mesh = plsc.VectorSubcoreMesh(
        core_axis_name="c", subcore_axis_name="s",
        num_cores=NC, num_subcores=NS,
    )

    @functools.partial(
        pl.kernel,
        out_type=jax.ShapeDtypeStruct((NUM_FIELDS, BATCH, EMB), jnp.float32),
        mesh=mesh,
        scratch_types=[
            pltpu.VMEM((N_CHUNKS, CHUNK), jnp.int32),
            pltpu.VMEM((B_PER_W, EMB), jnp.float32),
            pltpu.SemaphoreType.DMA,
        ],
        compiler_params=pltpu.CompilerParams(use_tc_tiling_on_sc=False),
    )
    def _sc_gather(
        f0, f1, f2, f3, f4, f5, f6, f7,
        t0, t1, t2, t3, t4, t5, t6, t7,
        out_hbm, idx_v, rows_v, sem,
    ):
        fields = [f0, f1, f2, f3, f4, f5, f6, f7]
        tables = [t0, t1, t2, t3, t4, t5, t6, t7]
        wid = lax.axis_index("s") * NC + lax.axis_index("c")
        base = wid * B_PER_W
        row_base = wid * N_CHUNKS
        for f in range(NUM_FIELDS):
            pltpu.sync_copy(fields[f].at[pl.ds(row_base, N_CHUNKS)], idx_v)
            copies = []
            for j in range(N_CHUNKS):
                copies.append(
                    pltpu.async_copy(
                        tables[f].at[idx_v.at[j]],
                        rows_v.at[pl.ds(j * CHUNK, CHUNK)],
                        sem,
                    )
                )
            for c in copies:
                c.wait()
            pltpu.sync_copy(rows_v, out_hbm.at[f, pl.ds(base, B_PER_W)])

    return _sc_gather


def _mm_body(g_ref, w_ref, b_ref, o_ref):
    acc = jnp.dot(g_ref[0], w_ref[0], preferred_element_type=jnp.float32)
    for i in range(1, NUM_FIELDS):
        acc += jnp.dot(g_ref[i], w_ref[i], preferred_element_type=jnp.float32)
    o_ref[...] = acc + b_ref[...]


BM = 2048

_tc_matmul = pl.pallas_call(
    _mm_body,
    grid=(BATCH // BM,),
    in_specs=[
        pl.BlockSpec((NUM_FIELDS, BM, EMB), lambda i: (0, i, 0)),
        pl.BlockSpec((NUM_FIELDS, EMB, ROUTING_DIM), lambda i: (0, 0, 0)),
        pl.BlockSpec((1, ROUTING_DIM), lambda i: (0, 0)),
    ],
    out_specs=pl.BlockSpec((BM, ROUTING_DIM), lambda i: (i, 0)),
    out_shape=jax.ShapeDtypeStruct((BATCH, ROUTING_DIM), jnp.float32),
)


def kernel(field_0, field_1, field_2, field_3, field_4, field_5, field_6,
           field_7, table_0, table_1, table_2, table_3, table_4, table_5,
           table_6, table_7, W, b):
    fields = [
        f.astype(jnp.int32).reshape(BATCH // CHUNK, CHUNK)
        for f in (field_0, field_1, field_2, field_3,
                  field_4, field_5, field_6, field_7)
    ]
    tables = (table_0, table_1, table_2, table_3,
              table_4, table_5, table_6, table_7)
    gathered = _make_sc_gather()(*fields, *tables)
    w3 = W.reshape(NUM_FIELDS, EMB, ROUTING_DIM)
    b2 = b.reshape(1, ROUTING_DIM)
    return _tc_matmul(gathered, w3, b2)
